# Initial kernel scaffold; baseline (speedup 1.0000x reference)
import jax
import jax.numpy as jnp
from jax.experimental import pallas as pl


def kernel(pred_mask, true_mask):
    raise NotImplementedError("write your pallas kernel here")


# speedup vs baseline: 2.4176x; 2.4176x over previous
import jax
import jax.numpy as jnp
from jax.experimental import pallas as pl
from jax.experimental.pallas import tpu as pltpu

NUM = 16
H = 1024
W = 1024
ROWS = 128
COLS = (H * W) // ROWS
BR = 16
GRID = ROWS // BR


def _hist_kernel(pred_ref, true_ref, out_ref, acc_ref, mse_ref):
    step = pl.program_id(0)

    @pl.when(step == 0)
    def _init():
        acc_ref[...] = jnp.zeros_like(acc_ref)
        mse_ref[...] = jnp.zeros_like(mse_ref)

    p = pred_ref[...]
    t = true_ref[...]
    d = p - t
    mse_ref[...] += jnp.reshape(jnp.sum(d * d), (1, 1))

    ids = (jax.lax.broadcasted_iota(jnp.int32, (BR * NUM, COLS), 0)
           & (NUM - 1)).astype(jnp.float32)
    pr = jnp.broadcast_to(p[:, None, :], (BR, NUM, COLS)).reshape(BR * NUM, COLS)
    tr = jnp.broadcast_to(t[:, None, :], (BR, NUM, COLS)).reshape(BR * NUM, COLS)
    ap = (pr == ids).astype(jnp.bfloat16)
    at = (tr == ids).astype(jnp.bfloat16)
    r = jax.lax.dot_general(ap, at, (((1,), (1,)), ((), ())),
                            preferred_element_type=jnp.float32)
    acc_ref[...] += r

    @pl.when(step == GRID - 1)
    def _fin():
        rm = acc_ref[...]
        s0 = jax.lax.broadcasted_iota(jnp.int32, (BR * NUM, BR * NUM), 0)
        s1 = jax.lax.broadcasted_iota(jnp.int32, (BR * NUM, BR * NUM), 1)
        rm = jnp.where((s0 >> 4) == (s1 >> 4), rm, 0.0)
        e0 = jax.lax.broadcasted_iota(jnp.int32, (BR * NUM, NUM), 0)
        e1 = jax.lax.broadcasted_iota(jnp.int32, (BR * NUM, NUM), 1)
        e = ((e0 & (NUM - 1)) == e1).astype(jnp.float32)
        re = jax.lax.dot_general(rm, e, (((1,), (0,)), ((), ())),
                                 preferred_element_type=jnp.float32)
        j = jax.lax.dot_general(e, re, (((0,), (0,)), ((), ())),
                                preferred_element_type=jnp.float32)

        ri = jax.lax.broadcasted_iota(jnp.int32, (NUM, NUM), 0)
        ci = jax.lax.broadcasted_iota(jnp.int32, (NUM, NUM), 1)
        valid = (ri >= 1) & (ci >= 1)
        inter = jnp.where(valid, j, 0.0)
        pc = jnp.sum(j, axis=1, keepdims=True)
        tc = jnp.sum(j, axis=0, keepdims=True)
        union = pc + tc - inter
        iou = jnp.where(valid & (union != 0.0),
                        inter / jnp.maximum(union, 1e-12), 0.0)
        max_p = jnp.max(iou, axis=1, keepdims=True)
        max_t = jnp.max(iou, axis=0, keepdims=True)
        rv = (jax.lax.broadcasted_iota(jnp.int32, (NUM, 1), 0) >= 1) & (pc > 0)
        cv = (jax.lax.broadcasted_iota(jnp.int32, (1, NUM), 1) >= 1) & (tc > 0)
        loss_p = jnp.sum(jnp.where(rv, 1.0 - max_p, 0.0))
        loss_t = jnp.sum(jnp.where(cv, 1.0 - max_t, 0.0))
        ninst = (jnp.sum(rv.astype(jnp.float32))
                 + jnp.sum(cv.astype(jnp.float32)))
        total = mse_ref[0, 0] / (H * W) / 1000.0 + loss_p + loss_t
        out_ref[...] = jnp.reshape(jnp.where(ninst == 0.0, 0.0, total), (1, 1))


def kernel(pred_mask, true_mask):
    p2 = pred_mask.reshape(ROWS, COLS)
    t2 = true_mask.reshape(ROWS, COLS)
    out = pl.pallas_call(
        _hist_kernel,
        grid=(GRID,),
        in_specs=[pl.BlockSpec((BR, COLS), lambda i: (i, 0)),
                  pl.BlockSpec((BR, COLS), lambda i: (i, 0))],
        out_specs=pl.BlockSpec((1, 1), lambda i: (0, 0)),
        out_shape=jax.ShapeDtypeStruct((1, 1), jnp.float32),
        scratch_shapes=[pltpu.VMEM((BR * NUM, BR * NUM), jnp.float32),
                        pltpu.VMEM((1, 1), jnp.float32)],
    )(p2, t2)
    return out[0, 0]
